# Initial kernel scaffold; baseline (speedup 1.0000x reference)
import jax
import jax.numpy as jnp
from jax.experimental import pallas as pl


def kernel(lt_u, lt_i, st_i, i_bias, W_ih, W_hh, b_ih, b_hh, users, users_items, pred_items):
    raise NotImplementedError("write your pallas kernel here")


# speedup vs baseline: 4.6285x; 4.6285x over previous
import functools

import jax
import jax.numpy as jnp
from jax import lax
from jax.experimental import pallas as pl
from jax.experimental.pallas import tpu as pltpu
from jax.experimental.pallas import tpu_sc as plsc

N_USERS = 100000
N_ITEMS = 1000000
H = 32
B = 4096
L = 50
P = 1
L_REG = 0.01
SCALE = 5.0
ALPHA = 0.5
N_NEG = 3

NC = 2
NS = 16
NW = NC * NS
CHUNK = 128
HIST = B * L
HIST_PER_W = HIST // NW
HIST_CHUNKS = HIST_PER_W // CHUNK
B_PER_W = B // NW

BLK = 512
NBLK = B // BLK


def _sc_gather_body(st_hbm, ltu_hbm, lti_hbm, bias_hbm,
                    hidx_hbm, users_hbm, pred_hbm,
                    x_out, ultu_out, ilti_out, ist_out, bias_out,
                    idx_v, rows_v, sidx_v, srows_v, brows_v, sem):
    wid = lax.axis_index("s") * NC + lax.axis_index("c")
    pltpu.sync_copy(hidx_hbm.at[pl.ds(wid * HIST_PER_W, HIST_PER_W)], idx_v)

    def chunk_step(j, carry):
        pltpu.async_copy(
            st_hbm.at[idx_v.at[pl.ds(j * CHUNK, CHUNK)]], rows_v, sem).wait()
        pltpu.sync_copy(
            rows_v, x_out.at[pl.ds(wid * HIST_PER_W + j * CHUNK, CHUNK)])
        return carry

    lax.fori_loop(0, HIST_CHUNKS, chunk_step, 0)

    out_base = wid * B_PER_W
    pltpu.sync_copy(users_hbm.at[pl.ds(out_base, B_PER_W)], sidx_v)
    pltpu.async_copy(ltu_hbm.at[sidx_v], srows_v, sem).wait()
    pltpu.sync_copy(srows_v, ultu_out.at[pl.ds(out_base, B_PER_W)])

    pltpu.sync_copy(pred_hbm.at[pl.ds(out_base, B_PER_W)], sidx_v)
    pltpu.async_copy(lti_hbm.at[sidx_v], srows_v, sem).wait()
    pltpu.sync_copy(srows_v, ilti_out.at[pl.ds(out_base, B_PER_W)])
    pltpu.async_copy(st_hbm.at[sidx_v], srows_v, sem).wait()
    pltpu.sync_copy(srows_v, ist_out.at[pl.ds(out_base, B_PER_W)])
    pltpu.async_copy(bias_hbm.at[sidx_v], brows_v, sem).wait()
    pltpu.sync_copy(brows_v, bias_out.at[pl.ds(out_base, B_PER_W)])


def _sc_gather(st_i, lt_u, lt_i, i_bias, hist_idx, users, pred):
    mesh = plsc.VectorSubcoreMesh(
        core_axis_name="c", subcore_axis_name="s",
        num_cores=NC, num_subcores=NS)
    f32 = jnp.float32
    out_type = (
        jax.ShapeDtypeStruct((HIST, H), f32),
        jax.ShapeDtypeStruct((B, H), f32),
        jax.ShapeDtypeStruct((B, H), f32),
        jax.ShapeDtypeStruct((B, H), f32),
        jax.ShapeDtypeStruct((B,), f32),
    )
    scratch = [
        pltpu.VMEM((HIST_PER_W,), jnp.int32),
        pltpu.VMEM((CHUNK, H), f32),
        pltpu.VMEM((CHUNK,), jnp.int32),
        pltpu.VMEM((CHUNK, H), f32),
        pltpu.VMEM((CHUNK,), f32),
        pltpu.SemaphoreType.DMA,
    ]
    run = pl.kernel(_sc_gather_body, out_type=out_type, mesh=mesh,
                    scratch_types=scratch,
                    compiler_params=pltpu.CompilerParams(
                        use_tc_tiling_on_sc=False))
    return run(st_i, lt_u, lt_i, i_bias, hist_idx, users, pred)


def _tc_body(x_ref, ultu_ref, ilti_ref, ist_ref, biasg_ref,
             wih_ref, whh_ref, bih_ref, bhh_ref,
             obj_ref, reg_ref, hid_ref, xw_ref, acc_ref):
    i = pl.program_id(0)
    f32 = jnp.float32

    @pl.when(i == 0)
    def _init():
        acc_ref[0] = 0.0
        acc_ref[1] = 0.0

    x2 = x_ref[...].reshape(L * BLK, H)
    b = bih_ref[...] + bhh_ref[...]
    xw = jnp.dot(x2, wih_ref[...], preferred_element_type=f32) + b
    xw_ref[...] = xw.reshape(L, BLK, H)

    whh = whh_ref[...]

    def step(t, h):
        return jnp.tanh(xw_ref[t] + jnp.dot(h, whh, preferred_element_type=f32))

    h = lax.fori_loop(0, L, step, jnp.zeros((BLK, H), f32))
    hid_ref[...] = h

    u_lt = ultu_ref[...]
    i_lt = ilti_ref[...]
    i_st = ist_ref[...]
    d1 = u_lt - i_lt
    lt_score = -jnp.sum(d1 * d1, axis=1, keepdims=True)
    d2 = h - i_st
    st_score = -jnp.sum(d2 * d2, axis=1, keepdims=True)
    score = (lt_score * ALPHA + st_score * (1.0 - ALPHA)) * SCALE + biasg_ref[...]

    row = lax.broadcasted_iota(jnp.int32, (BLK, 1), 0)
    is_pos = (row % (N_NEG + 1)) == 0
    loss_sum = jnp.zeros((), f32)
    for m in range(1, N_NEG + 1):
        neg = pltpu.roll(score, BLK - m, 0)
        z = neg - score
        sp = jnp.maximum(z, 0.0) + jnp.log1p(jnp.exp(-jnp.abs(z)))
        loss_sum = loss_sum + jnp.sum(jnp.where(is_pos, sp, 0.0))

    bias_g = biasg_ref[...]
    reg_sum = (jnp.sum(u_lt * u_lt) / (B * H)
               + jnp.sum(i_lt * i_lt) / (B * H)
               + jnp.sum(i_st * i_st) / (B * H)
               + jnp.sum(bias_g * bias_g) / B)

    acc_ref[0] += loss_sum
    acc_ref[1] += reg_sum

    @pl.when(i == NBLK - 1)
    def _fin():
        obj_ref[0, 0] = acc_ref[0] / (N_NEG * (B // (N_NEG + 1)) * P)
        reg_ref[0, 0] = acc_ref[1] * L_REG


def _tc_compute(x, u_lt, i_lt, i_st, bias_g, W_ih, W_hh, b_ih, b_hh):
    f32 = jnp.float32
    grid = (NBLK,)
    out_shape = (
        jax.ShapeDtypeStruct((1, 1), f32),
        jax.ShapeDtypeStruct((1, 1), f32),
        jax.ShapeDtypeStruct((B, H), f32),
    )
    in_specs = [
        pl.BlockSpec((L, BLK, H), lambda i: (0, i, 0)),
        pl.BlockSpec((BLK, H), lambda i: (i, 0)),
        pl.BlockSpec((BLK, H), lambda i: (i, 0)),
        pl.BlockSpec((BLK, H), lambda i: (i, 0)),
        pl.BlockSpec((BLK, 1), lambda i: (i, 0)),
        pl.BlockSpec((H, H), lambda i: (0, 0)),
        pl.BlockSpec((H, H), lambda i: (0, 0)),
        pl.BlockSpec((1, H), lambda i: (0, 0)),
        pl.BlockSpec((1, H), lambda i: (0, 0)),
    ]
    out_specs = (
        pl.BlockSpec(memory_space=pltpu.SMEM, index_map=lambda i: (0, 0)),
        pl.BlockSpec(memory_space=pltpu.SMEM, index_map=lambda i: (0, 0)),
        pl.BlockSpec((BLK, H), lambda i: (i, 0)),
    )
    return pl.pallas_call(
        _tc_body,
        grid=grid,
        in_specs=in_specs,
        out_specs=out_specs,
        out_shape=out_shape,
        scratch_shapes=[
            pltpu.VMEM((L, BLK, H), f32),
            pltpu.SMEM((2,), f32),
        ],
    )(x, u_lt, i_lt, i_st, bias_g, W_ih, W_hh, b_ih, b_hh)


def kernel(lt_u, lt_i, st_i, i_bias, W_ih, W_hh, b_ih, b_hh,
           users, users_items, pred_items):
    hist_idx = users_items.T.reshape(HIST)
    pred1 = pred_items.reshape(B)

    x, u_lt, i_lt, i_st, bias_g = _sc_gather(
        st_i, lt_u, lt_i, i_bias.reshape(N_ITEMS), hist_idx, users, pred1)

    obj, reg, hidden = _tc_compute(
        x.reshape(L, B, H), u_lt, i_lt, i_st, bias_g.reshape(B, 1),
        W_ih, W_hh, b_ih.reshape(1, H), b_hh.reshape(1, H))
    return obj.reshape(()), reg.reshape(()), hidden
